# Initial kernel scaffold; baseline (speedup 1.0000x reference)
import jax
import jax.numpy as jnp
from jax.experimental import pallas as pl


def kernel(x, router_w, W1, b1, W2, b2):
    raise NotImplementedError("write your pallas kernel here")


# speedup vs baseline: 2.7401x; 2.7401x over previous
import functools

import jax
import jax.numpy as jnp
from jax.experimental import pallas as pl
from jax.experimental.pallas import tpu as pltpu

N_EXPERTS = 8
D_MODEL = 1024
D_FF = 2048
TOP_P = 0.9
AUX_COEFF = 0.01


def _routing_kernel(x_ref, rw_ref, we_ref, aux_ref):
    x = x_ref[...]
    rw = rw_ref[...]
    logits = jax.lax.dot_general(rw, x, (((1,), (1,)), ((), ())),
                                 preferred_element_type=jnp.float32)
    m = jnp.max(logits, axis=0, keepdims=True)
    ex = jnp.exp(logits - m)
    probs = ex / jnp.sum(ex, axis=0, keepdims=True)

    eidx = jax.lax.broadcasted_iota(jnp.int32, probs.shape, 0)
    m1 = jnp.max(probs, axis=0, keepdims=True)
    i1 = jnp.min(jnp.where(probs == m1, eidx, N_EXPERTS), axis=0, keepdims=True)
    probs2 = jnp.where(eidx == i1, -1.0, probs)
    m2 = jnp.max(probs2, axis=0, keepdims=True)
    i2 = jnp.min(jnp.where(probs2 == m2, eidx, N_EXPERTS), axis=0, keepdims=True)

    keep2 = m1 < TOP_P
    w1 = m1
    w2 = jnp.where(keep2, m2, 0.0)
    denom = jnp.maximum(w1 + w2, 1e-9)
    w1n = w1 / denom
    w2n = jnp.where(keep2, w2 / denom, 0.0)

    sel1 = (eidx == i1).astype(jnp.float32)
    sel2 = ((eidx == i2) & keep2).astype(jnp.float32)
    we = w1n * sel1 + w2n * sel2
    we_ref[...] = we.T

    counts = jnp.sum(sel1 + sel2, axis=1, keepdims=True)
    total = jnp.maximum(jnp.sum(counts), 1.0)
    p_mean = jnp.mean(probs, axis=1, keepdims=True)
    aux = N_EXPERTS * jnp.sum((counts / total) * p_mean, keepdims=True)
    aux_ref[...] = AUX_COEFF * aux.reshape(1, 1)


def _expert_kernel(we_ref, x_ref, w1_ref, b1_ref, w2_ref, b2_ref, out_ref):
    e = pl.program_id(1)
    x = x_ref[...]
    h = jax.lax.dot_general(x, w1_ref[0], (((1,), (1,)), ((), ())),
                            preferred_element_type=jnp.float32)
    h = h + b1_ref[0]
    h = 0.5 * h * (1.0 + jax.lax.erf(h * (2.0 ** -0.5)))
    o = jax.lax.dot_general(h, w2_ref[0], (((1,), (1,)), ((), ())),
                            preferred_element_type=jnp.float32)
    o = o + b2_ref[0]
    eidx = jax.lax.broadcasted_iota(jnp.int32, (we_ref.shape[0], N_EXPERTS), 1)
    scale = jnp.sum(we_ref[...] * (eidx == e).astype(jnp.float32),
                    axis=1, keepdims=True)
    o = o * scale

    @pl.when(e == 0)
    def _():
        out_ref[...] = o

    @pl.when(e != 0)
    def _():
        out_ref[...] = out_ref[...] + o


def kernel(x, router_w, W1, b1, W2, b2):
    Bv, Tv, d_model = x.shape
    N = Bv * Tv
    x_flat = x.reshape(N, d_model)

    we, aux = pl.pallas_call(
        _routing_kernel,
        out_shape=[
            jax.ShapeDtypeStruct((N, N_EXPERTS), jnp.float32),
            jax.ShapeDtypeStruct((1, 1), jnp.float32),
        ],
    )(x_flat, router_w)

    BT = 512
    grid = (N // BT, N_EXPERTS)
    out = pl.pallas_call(
        _expert_kernel,
        grid=grid,
        in_specs=[
            pl.BlockSpec((BT, N_EXPERTS), lambda t, e: (t, 0)),
            pl.BlockSpec((BT, d_model), lambda t, e: (t, 0)),
            pl.BlockSpec((1, D_FF, d_model), lambda t, e: (e, 0, 0)),
            pl.BlockSpec((1, 1, D_FF), lambda t, e: (e, 0, 0)),
            pl.BlockSpec((1, d_model, D_FF), lambda t, e: (e, 0, 0)),
            pl.BlockSpec((1, 1, d_model), lambda t, e: (e, 0, 0)),
        ],
        out_specs=pl.BlockSpec((BT, d_model), lambda t, e: (t, 0)),
        out_shape=jax.ShapeDtypeStruct((N, d_model), jnp.float32),
    )(we, x_flat, W1, b1.reshape(N_EXPERTS, 1, D_FF), W2,
      b2.reshape(N_EXPERTS, 1, d_model))

    return out.reshape(Bv, Tv, d_model), aux.reshape(())
